# Initial kernel scaffold; baseline (speedup 1.0000x reference)
import jax
import jax.numpy as jnp
from jax.experimental import pallas as pl


def kernel(indices, table, gamma, beta):
    raise NotImplementedError("write your pallas kernel here")


# speedup vs baseline: 3.7227x; 3.7227x over previous
import functools

import jax
import jax.numpy as jnp
from jax import lax
from jax.experimental import pallas as pl
from jax.experimental.pallas import tpu as pltpu
from jax.experimental.pallas import tpu_sc as plsc

_VOCAB = 1000
_EMB = 128
_B = 16384
_L = 20
_EPS = 1e-5

_NC = 2
_NS = 16
_NW = _NC * _NS
_TOK = _B * _L
_PER_W = _TOK // _NW
_C = 128
_NCHUNK = _PER_W // _C
_NBUF = 2


def _ln_table_kernel(table_ref, gamma_ref, beta_ref, out_ref):
    x = table_ref[...]
    mean = jnp.mean(x, axis=1, keepdims=True)
    c = x - mean
    var = jnp.mean(c * c, axis=1, keepdims=True)
    out_ref[...] = c * lax.rsqrt(var + _EPS) * gamma_ref[...] + beta_ref[...]


@functools.partial(
    pl.kernel,
    out_type=jax.ShapeDtypeStruct((_TOK, _EMB), jnp.float32),
    mesh=plsc.VectorSubcoreMesh(core_axis_name="c", subcore_axis_name="s"),
    scratch_types=[
        pltpu.VMEM((_NCHUNK, _C), jnp.int32),
        pltpu.VMEM((_C, _EMB), jnp.float32),
        pltpu.VMEM((_C, _EMB), jnp.float32),
        pltpu.SemaphoreType.DMA,
        pltpu.SemaphoreType.DMA,
    ],
)
def _sc_gather(ntable_hbm, idx_hbm, out_hbm, idx_v, buf0, buf1, sem0, sem1):
    wid = lax.axis_index("s") * _NC + lax.axis_index("c")
    base = wid * _PER_W
    pltpu.sync_copy(idx_hbm.at[wid], idx_v)

    bufs = (buf0, buf1)
    sems = (sem0, sem1)

    for b in range(_NBUF):
        pltpu.async_copy(ntable_hbm.at[idx_v.at[b]], bufs[b], sems[b])

    def body(i, carry):
        for b in range(_NBUF):
            g = i * _NBUF + b
            pltpu.make_async_copy(
                ntable_hbm.at[idx_v.at[g]], bufs[b], sems[b]
            ).wait()
            pltpu.sync_copy(bufs[b], out_hbm.at[pl.ds(base + g * _C, _C)])

            @pl.when(g + _NBUF < _NCHUNK)
            def _():
                pltpu.async_copy(
                    ntable_hbm.at[idx_v.at[g + _NBUF]], bufs[b], sems[b]
                )

        return carry

    lax.fori_loop(0, _NCHUNK // _NBUF, body, 0)


def kernel(indices, table, gamma, beta):
    ntable = pl.pallas_call(
        _ln_table_kernel,
        out_shape=jax.ShapeDtypeStruct((_VOCAB, _EMB), jnp.float32),
    )(table, gamma.reshape(1, _EMB), beta.reshape(1, _EMB))

    idx = indices.astype(jnp.int32).reshape(_NW, _NCHUNK, _C)
    out = _sc_gather(ntable, idx)
    return out.reshape(_B, _L, _EMB)


# speedup vs baseline: 5.8984x; 1.5844x over previous
import functools

import jax
import jax.numpy as jnp
from jax import lax
from jax.experimental import pallas as pl
from jax.experimental.pallas import tpu as pltpu
from jax.experimental.pallas import tpu_sc as plsc

_VOCAB = 1000
_EMB = 128
_B = 16384
_L = 20
_EPS = 1e-5

_NC = 2
_NS = 16
_NW = _NC * _NS
_BPW = _B // _NW
_RB = 4
_NCHUNK = _BPW // _RB
_NBUF = 4
_AHEAD = 2


def _ln_table_kernel(table_ref, gamma_ref, beta_ref, out_ref):
    x = table_ref[...]
    mean = jnp.mean(x, axis=1, keepdims=True)
    c = x - mean
    var = jnp.mean(c * c, axis=1, keepdims=True)
    out_ref[...] = c * lax.rsqrt(var + _EPS) * gamma_ref[...] + beta_ref[...]


@functools.partial(
    pl.kernel,
    out_type=jax.ShapeDtypeStruct((_B, _L, _EMB), jnp.float32),
    mesh=plsc.VectorSubcoreMesh(core_axis_name="c", subcore_axis_name="s"),
    scratch_types=[
        pltpu.VMEM((_BPW, _L), jnp.int32),
        pltpu.VMEM((_NBUF, _RB, _L, _EMB), jnp.float32),
        pltpu.SemaphoreType.DMA((_NBUF,)),
        pltpu.SemaphoreType.DMA((_NBUF,)),
    ],
)
def _sc_gather(ntable_hbm, idx_hbm, out_hbm, idx_v, bufs, gsems, wsems):
    wid = lax.axis_index("s") * _NC + lax.axis_index("c")
    base = wid * _BPW
    pltpu.sync_copy(idx_hbm.at[wid], idx_v)

    def fire_gathers(m, k):
        for j in range(_RB):
            r = m * _RB + j
            pltpu.async_copy(
                ntable_hbm.at[idx_v.at[r]], bufs.at[k, j], gsems.at[k]
            )

    def drain_gathers(m, k):
        for j in range(_RB):
            r = m * _RB + j
            pltpu.make_async_copy(
                ntable_hbm.at[idx_v.at[r]], bufs.at[k, j], gsems.at[k]
            ).wait()

    def fire_write(m, k):
        pltpu.async_copy(
            bufs.at[k], out_hbm.at[pl.ds(base + m * _RB, _RB)], wsems.at[k]
        )

    def wait_write(m, k):
        pltpu.make_async_copy(
            bufs.at[k], out_hbm.at[pl.ds(base + m * _RB, _RB)], wsems.at[k]
        ).wait()

    for m in range(_AHEAD):
        fire_gathers(m, m % _NBUF)

    def body(m, carry):
        k = m % _NBUF
        drain_gathers(m, k)
        fire_write(m, k)

        nxt = m + _AHEAD
        kn = nxt % _NBUF

        @pl.when(nxt < _NCHUNK)
        def _():
            @pl.when(nxt >= _NBUF)
            def _():
                wait_write(nxt - _NBUF, kn)

            fire_gathers(nxt, kn)

        return carry

    lax.fori_loop(0, _NCHUNK, body, 0)

    for t in range(_NBUF):
        m = _NCHUNK - _NBUF + t
        wait_write(m, m % _NBUF)


def kernel(indices, table, gamma, beta):
    ntable = pl.pallas_call(
        _ln_table_kernel,
        out_shape=jax.ShapeDtypeStruct((_VOCAB, _EMB), jnp.float32),
    )(table, gamma.reshape(1, _EMB), beta.reshape(1, _EMB))

    idx = indices.astype(jnp.int32).reshape(_NW, _BPW, _L)
    return _sc_gather(ntable, idx)


# speedup vs baseline: 9.3546x; 1.5860x over previous
import functools

import jax
import jax.numpy as jnp
from jax import lax
from jax.experimental import pallas as pl
from jax.experimental.pallas import tpu as pltpu
from jax.experimental.pallas import tpu_sc as plsc

_VOCAB = 1000
_EMB = 128
_B = 16384
_L = 20
_EPS = 1e-5

_NC = 2
_NS = 16
_NW = _NC * _NS
_BPW = _B // _NW
_RB = 4
_NCHUNK = _BPW // _RB
_NBUF = 4
_AHEAD = 2


def _ln_table_kernel(table_ref, gamma_ref, beta_ref, out_ref):
    x = table_ref[...]
    mean = jnp.mean(x, axis=1, keepdims=True)
    c = x - mean
    var = jnp.mean(c * c, axis=1, keepdims=True)
    out_ref[...] = c * lax.rsqrt(var + _EPS) * gamma_ref[...] + beta_ref[...]


@functools.partial(
    pl.kernel,
    out_type=jax.ShapeDtypeStruct((_B, _L, _EMB), jnp.float32),
    mesh=plsc.VectorSubcoreMesh(core_axis_name="c", subcore_axis_name="s"),
    scratch_types=[
        pltpu.VMEM((_BPW, _L), jnp.int32),
        pltpu.VMEM((_NBUF, _RB, _L, _EMB), jnp.float32),
        pltpu.VMEM_SHARED((_VOCAB, _EMB), jnp.float32),
        pltpu.SemaphoreType.DMA((_NBUF,)),
        pltpu.SemaphoreType.DMA((_NBUF,)),
    ],
)
def _sc_gather(ntable_hbm, idx_hbm, out_hbm, idx_v, bufs, ntable_sp, gsems, wsems):
    sid = lax.axis_index("s")
    wid = sid * _NC + lax.axis_index("c")
    base = wid * _BPW
    @pl.when(sid == 0)
    def _():
        pltpu.sync_copy(ntable_hbm, ntable_sp)

    pltpu.sync_copy(idx_hbm.at[wid], idx_v)
    plsc.subcore_barrier()

    def fire_gathers(m, k):
        for j in range(_RB):
            r = m * _RB + j
            pltpu.async_copy(
                ntable_sp.at[idx_v.at[r]], bufs.at[k, j], gsems.at[k]
            )

    def drain_gathers(m, k):
        for j in range(_RB):
            r = m * _RB + j
            pltpu.make_async_copy(
                ntable_sp.at[idx_v.at[r]], bufs.at[k, j], gsems.at[k]
            ).wait()

    def fire_write(m, k):
        pltpu.async_copy(
            bufs.at[k], out_hbm.at[pl.ds(base + m * _RB, _RB)], wsems.at[k]
        )

    def wait_write(m, k):
        pltpu.make_async_copy(
            bufs.at[k], out_hbm.at[pl.ds(base + m * _RB, _RB)], wsems.at[k]
        ).wait()

    for m in range(_AHEAD):
        fire_gathers(m, m % _NBUF)

    def body(m, carry):
        k = m % _NBUF
        drain_gathers(m, k)
        fire_write(m, k)

        nxt = m + _AHEAD
        kn = nxt % _NBUF

        @pl.when(nxt < _NCHUNK)
        def _():
            @pl.when(nxt >= _NBUF)
            def _():
                wait_write(nxt - _NBUF, kn)

            fire_gathers(nxt, kn)

        return carry

    lax.fori_loop(0, _NCHUNK, body, 0)

    for t in range(_NBUF):
        m = _NCHUNK - _NBUF + t
        wait_write(m, m % _NBUF)


def kernel(indices, table, gamma, beta):
    ntable = pl.pallas_call(
        _ln_table_kernel,
        out_shape=jax.ShapeDtypeStruct((_VOCAB, _EMB), jnp.float32),
    )(table, gamma.reshape(1, _EMB), beta.reshape(1, _EMB))

    idx = indices.astype(jnp.int32).reshape(_NW, _BPW, _L)
    return _sc_gather(ntable, idx)
